# Initial kernel scaffold; baseline (speedup 1.0000x reference)
import functools

import jax
import jax.numpy as jnp
from jax import lax
from jax.experimental import pallas as pl
from jax.experimental.pallas import tpu as pltpu
from jax.experimental.pallas import tpu_sc as plsc

N_USERS = 25000
N_ITEMS = 25000
N_NODES = N_USERS + N_ITEMS
N_EDGES = 800000
D = 64

NC = 2
NS = 16
L = 16

HALF = N_NODES // NC
ACC_ROWS = HALF + 8
DUMP = HALF

CHUNK = 512
SUB = 128
NSUB = CHUNK // SUB
E_PER_TILE = 51200
E_PAD = E_PER_TILE * NS
N_CHUNKS = E_PER_TILE // CHUNK

ZROWS = ACC_ROWS // NS
W_HI = 1563
W_LO = 1562


def _hop_kernel(x_hbm, rows_hbm, cols_hbm, vals_hbm, y_hbm,
                gbuf, cols2, lidx2, rows_v, vals_v, sem):
    c = lax.axis_index("c")
    s = lax.axis_index("s")

    def body(acc):
        def zrow(r, _):
            for q in range(D // L):
                gbuf[r, pl.ds(q * L, L)] = jnp.zeros((L,), jnp.float32)
            return 0
        lax.fori_loop(0, CHUNK, zrow, 0)

        zstart = s * ZROWS
        for k in range(ZROWS // CHUNK):
            pltpu.sync_copy(gbuf, acc.at[pl.ds(zstart + k * CHUNK, CHUNK)])
        zrem = ZROWS % CHUNK
        if zrem:
            pltpu.sync_copy(gbuf.at[pl.ds(0, zrem)],
                            acc.at[pl.ds(zstart + (ZROWS // CHUNK) * CHUNK, zrem)])
        plsc.subcore_barrier()

        base_half = c * HALF
        tile_base = s * E_PER_TILE
        tile_crow = s * (E_PER_TILE // SUB)

        def chunk_step(ci, _):
            e_base = tile_base + ci * CHUNK
            pltpu.sync_copy(rows_hbm.at[pl.ds(e_base, CHUNK)], rows_v)
            pltpu.sync_copy(vals_hbm.at[pl.ds(e_base, CHUNK)], vals_v)
            pltpu.sync_copy(cols_hbm.at[pl.ds(tile_crow + ci * NSUB, NSUB)],
                            cols2)
            copies = [pltpu.async_copy(x_hbm.at[cols2.at[k]],
                                       gbuf.at[pl.ds(k * SUB, SUB)], sem)
                      for k in range(NSUB)]
            for j in range(CHUNK // L):
                r16 = rows_v[pl.ds(j * L, L)]
                l16 = r16 - base_half
                oob = (l16 < 0) | (l16 >= HALF)
                l16 = jnp.where(oob, DUMP, l16)
                lidx2[j // (SUB // L), pl.ds((j % (SUB // L)) * L, L)] = l16
            for cp in copies:
                cp.wait()
            def scale_row(r, _):
                v = vals_v[r]
                for q in range(D // L):
                    gbuf[r, pl.ds(q * L, L)] = gbuf[r, pl.ds(q * L, L)] * v
                return 0
            lax.fori_loop(0, CHUNK, scale_row, 0)
            for k in range(NSUB):
                pltpu.sync_copy(gbuf.at[pl.ds(k * SUB, SUB)],
                                acc.at[lidx2.at[k]], add=True)
            return 0

        lax.fori_loop(0, N_CHUNKS, chunk_step, 0)
        plsc.subcore_barrier()

        wstart_hi = s * W_HI
        wstart_lo = s * W_LO + 8

        @pl.when(s < 8)
        def _():
            for k in range(W_HI // CHUNK):
                off = wstart_hi + k * CHUNK
                pltpu.sync_copy(acc.at[pl.ds(off, CHUNK)],
                                y_hbm.at[pl.ds(base_half + off, CHUNK)])
            toff = wstart_hi + (W_HI // CHUNK) * CHUNK
            pltpu.sync_copy(acc.at[pl.ds(toff, W_HI % CHUNK)],
                            y_hbm.at[pl.ds(base_half + toff, W_HI % CHUNK)])

        @pl.when(s >= 8)
        def _():
            for k in range(W_LO // CHUNK):
                off = wstart_lo + k * CHUNK
                pltpu.sync_copy(acc.at[pl.ds(off, CHUNK)],
                                y_hbm.at[pl.ds(base_half + off, CHUNK)])
            toff = wstart_lo + (W_LO // CHUNK) * CHUNK
            pltpu.sync_copy(acc.at[pl.ds(toff, W_LO % CHUNK)],
                            y_hbm.at[pl.ds(base_half + toff, W_LO % CHUNK)])

    pl.run_scoped(body, pltpu.VMEM_SHARED((ACC_ROWS, D), jnp.float32))


_mesh = plsc.VectorSubcoreMesh(core_axis_name="c", subcore_axis_name="s")

_hop = functools.partial(
    pl.kernel,
    mesh=_mesh,
    out_type=jax.ShapeDtypeStruct((N_NODES, D), jnp.float32),
    scratch_types=[
        pltpu.VMEM((CHUNK, D), jnp.float32),
        pltpu.VMEM((NSUB, SUB), jnp.int32),
        pltpu.VMEM((NSUB, SUB), jnp.int32),
        pltpu.VMEM((CHUNK,), jnp.int32),
        pltpu.VMEM((CHUNK,), jnp.float32),
        pltpu.SemaphoreType.DMA,
    ],
)(_hop_kernel)


def kernel(user_embed, item_embed, edge_rows, edge_cols, edge_vals):
    x0 = jnp.concatenate([user_embed, item_embed], axis=0)
    pad = E_PAD - N_EDGES
    rows_p = jnp.concatenate([edge_rows, jnp.zeros((pad,), jnp.int32)])
    cols_p = jnp.concatenate([edge_cols, jnp.zeros((pad,), jnp.int32)])
    vals_p = jnp.concatenate([edge_vals, jnp.zeros((pad,), jnp.float32)])
    cols_r = cols_p.reshape(E_PAD // SUB, SUB)

    x1 = _hop(x0, rows_p, cols_r, vals_p)
    x2 = _hop(x1, rows_p, cols_r, vals_p)
    x3 = _hop(x2, rows_p, cols_r, vals_p)
    embs = jnp.stack([x0, x1, x2, x3], axis=1)
    return (embs[:N_USERS], embs[N_USERS:])


# speedup vs baseline: 1.9061x; 1.9061x over previous
import functools

import jax
import jax.numpy as jnp
from jax import lax
from jax.experimental import pallas as pl
from jax.experimental.pallas import tpu as pltpu
from jax.experimental.pallas import tpu_sc as plsc

N_USERS = 25000
N_ITEMS = 25000
N_NODES = N_USERS + N_ITEMS
N_EDGES = 800000
D = 64

NC = 2
NS = 16
L = 16

HALF = N_NODES // NC
WPT = 1568
ACC_ROWS = WPT * NS
DUMP = HALF + 80
W_LAST = HALF - 15 * WPT

CHUNK = 256
SUB = 128
NSUB = CHUNK // SUB
E_PER_TILE = 51200
E_PAD = E_PER_TILE * NS
N_CHUNKS = E_PER_TILE // CHUNK


def _hop_kernel(x_hbm, rows_hbm, cols_hbm, vals_hbm, y_hbm,
                acc, gbuf, cols_v, lidx2, rows_v, vals_v, sem):
    c = lax.axis_index("c")
    s = lax.axis_index("s")

    if True:
        def zrow(r, _):
            for q in range(D // L):
                gbuf[r, pl.ds(q * L, L)] = jnp.zeros((L,), jnp.float32)
            return 0
        lax.fori_loop(0, CHUNK, zrow, 0)

        zstart = s * WPT
        for k in range(WPT // CHUNK):
            pltpu.sync_copy(gbuf, acc.at[pl.ds(zstart + k * CHUNK, CHUNK)])
        zrem = WPT % CHUNK
        pltpu.sync_copy(gbuf.at[pl.ds(0, zrem)],
                        acc.at[pl.ds(zstart + (WPT // CHUNK) * CHUNK, zrem)])
        plsc.subcore_barrier()

        base_half = c * HALF
        tile_base = s * E_PER_TILE

        def chunk_step(ci, _):
            e_base = tile_base + ci * CHUNK
            pltpu.sync_copy(rows_hbm.at[pl.ds(e_base, CHUNK)], rows_v)
            pltpu.sync_copy(vals_hbm.at[pl.ds(e_base, CHUNK)], vals_v)
            pltpu.sync_copy(cols_hbm.at[pl.ds(e_base, CHUNK)], cols_v)
            copies = [pltpu.async_copy(x_hbm.at[cols_v.at[pl.ds(k * SUB, SUB)]],
                                       gbuf.at[pl.ds(k * SUB, SUB)], sem)
                      for k in range(NSUB)]
            for j in range(CHUNK // L):
                r16 = rows_v[pl.ds(j * L, L)]
                l16 = r16 - base_half
                oob = (l16 < 0) | (l16 >= HALF)
                l16 = jnp.where(oob, DUMP, l16)
                lidx2[j // (SUB // L), pl.ds((j % (SUB // L)) * L, L)] = l16
            for cp in copies:
                cp.wait()
            def scale_rows(g, _):
                r0 = g * L
                v16 = vals_v[pl.ds(r0, L)]
                for i in range(L):
                    v = v16[i]
                    for q in range(D // L):
                        gbuf[r0 + i, pl.ds(q * L, L)] = (
                            gbuf[r0 + i, pl.ds(q * L, L)] * v)
                return 0
            lax.fori_loop(0, CHUNK // L, scale_rows, 0)
            for k in range(NSUB):
                pltpu.sync_copy(gbuf.at[pl.ds(k * SUB, SUB)],
                                acc.at[lidx2.at[k]], add=True)
            return 0

        lax.fori_loop(0, N_CHUNKS, chunk_step, 0)
        plsc.subcore_barrier()

        wstart = s * WPT

        @pl.when(s < NS - 1)
        def _():
            for k in range(WPT // CHUNK):
                off = wstart + k * CHUNK
                pltpu.sync_copy(acc.at[pl.ds(off, CHUNK)],
                                y_hbm.at[pl.ds(base_half + off, CHUNK)])
            toff = wstart + (WPT // CHUNK) * CHUNK
            pltpu.sync_copy(acc.at[pl.ds(toff, WPT % CHUNK)],
                            y_hbm.at[pl.ds(base_half + toff, WPT % CHUNK)])

        @pl.when(s == NS - 1)
        def _():
            for k in range(W_LAST // CHUNK):
                off = wstart + k * CHUNK
                pltpu.sync_copy(acc.at[pl.ds(off, CHUNK)],
                                y_hbm.at[pl.ds(base_half + off, CHUNK)])
            toff = wstart + (W_LAST // CHUNK) * CHUNK
            pltpu.sync_copy(acc.at[pl.ds(toff, W_LAST % CHUNK)],
                            y_hbm.at[pl.ds(base_half + toff, W_LAST % CHUNK)])


_mesh = plsc.VectorSubcoreMesh(core_axis_name="c", subcore_axis_name="s")

_hop = functools.partial(
    pl.kernel,
    mesh=_mesh,
    compiler_params=pltpu.CompilerParams(use_tc_tiling_on_sc=False),
    out_type=jax.ShapeDtypeStruct((N_NODES, D), jnp.float32),
    scratch_types=[
        pltpu.VMEM_SHARED((ACC_ROWS, D), jnp.float32),
        pltpu.VMEM((CHUNK, D), jnp.float32),
        pltpu.VMEM((CHUNK,), jnp.int32),
        pltpu.VMEM((NSUB, SUB), jnp.int32),
        pltpu.VMEM((CHUNK,), jnp.int32),
        pltpu.VMEM((CHUNK,), jnp.float32),
        pltpu.SemaphoreType.DMA,
    ],
)(_hop_kernel)


def kernel(user_embed, item_embed, edge_rows, edge_cols, edge_vals):
    x0 = jnp.concatenate([user_embed, item_embed], axis=0)
    pad = E_PAD - N_EDGES
    rows_p = jnp.concatenate([edge_rows, jnp.zeros((pad,), jnp.int32)])
    cols_p = jnp.concatenate([edge_cols, jnp.zeros((pad,), jnp.int32)])
    vals_p = jnp.concatenate([edge_vals, jnp.zeros((pad,), jnp.float32)])

    x1 = _hop(x0, rows_p, cols_p, vals_p)
    x2 = _hop(x1, rows_p, cols_p, vals_p)
    x3 = _hop(x2, rows_p, cols_p, vals_p)
    embs = jnp.stack([x0, x1, x2, x3], axis=1)
    return (embs[:N_USERS], embs[N_USERS:])


# speedup vs baseline: 2.8226x; 1.4808x over previous
import functools

import jax
import jax.numpy as jnp
from jax import lax
from jax.experimental import pallas as pl
from jax.experimental.pallas import tpu as pltpu
from jax.experimental.pallas import tpu_sc as plsc

N_USERS = 25000
N_ITEMS = 25000
N_NODES = N_USERS + N_ITEMS
N_EDGES = 800000
D = 64

NC = 2
NS = 16
L = 16

HALF = N_NODES // NC
WPT = 1568
ACC_ROWS = WPT * NS
DUMP = HALF
W_LAST = HALF - 15 * WPT

CHUNK = 192
SUB = 96
NSUB = CHUNK // SUB
GROUPS = CHUNK // L
N_CHUNKS = 268
E_PER_TILE = N_CHUNKS * CHUNK
E_ALLOC = NS * E_PER_TILE + 2 * CHUNK
OUTER = N_CHUNKS // 4


def _hop_kernel(x_hbm, rows_hbm, cols_hbm, vals_hbm, y_hbm,
                acc, gbuf, rows4, cols4, vals4, lidx,
                sem_e, sem_g, sem_s):
    c = lax.axis_index("c")
    s = lax.axis_index("s")
    base_half = c * HALF
    tile_base = s * E_PER_TILE

    def fire_edges(i, jj):
        eb = tile_base + i * CHUNK
        pltpu.async_copy(rows_hbm.at[pl.ds(eb, CHUNK)], rows4.at[jj], sem_e)
        pltpu.async_copy(cols_hbm.at[pl.ds(eb, CHUNK)], cols4.at[jj], sem_e)
        pltpu.async_copy(vals_hbm.at[pl.ds(eb, CHUNK)], vals4.at[jj], sem_e)

    def wait_edges(jj):
        for ref in (rows4, cols4, vals4):
            pltpu.make_async_copy(rows_hbm.at[pl.ds(0, CHUNK)],
                                  ref.at[jj], sem_e).wait()

    def fire_gather(jj, b):
        for k in range(NSUB):
            pltpu.async_copy(
                x_hbm.at[cols4.at[jj, pl.ds(k * SUB, SUB)]],
                gbuf.at[b, pl.ds(k * SUB, SUB)], sem_g)

    def wait_gather(jj, b):
        for k in range(NSUB):
            pltpu.make_async_copy(
                x_hbm.at[cols4.at[jj, pl.ds(k * SUB, SUB)]],
                gbuf.at[b, pl.ds(k * SUB, SUB)], sem_g).wait()

    def fire_scatter(b):
        for k in range(NSUB):
            pltpu.async_copy(gbuf.at[b, pl.ds(k * SUB, SUB)],
                             acc.at[lidx.at[b, k]], sem_s, add=True)

    def wait_scatter(b):
        for k in range(NSUB):
            pltpu.make_async_copy(gbuf.at[b, pl.ds(k * SUB, SUB)],
                                  acc.at[lidx.at[b, k]], sem_s).wait()

    fire_edges(0, 0)
    fire_edges(1, 1)

    def zrow(r, _):
        for q in range(D // L):
            gbuf[0, r, pl.ds(q * L, L)] = jnp.zeros((L,), jnp.float32)
        return 0
    lax.fori_loop(0, CHUNK, zrow, 0)
    dump16 = jnp.full((L,), DUMP, jnp.int32)
    for b in range(2):
        for k in range(NSUB):
            for m in range(SUB // L):
                lidx[b, k, pl.ds(m * L, L)] = dump16

    zstart = s * WPT
    for k in range(WPT // CHUNK):
        pltpu.sync_copy(gbuf.at[0], acc.at[pl.ds(zstart + k * CHUNK, CHUNK)])
    zrem = WPT % CHUNK
    pltpu.sync_copy(gbuf.at[0, pl.ds(0, zrem)],
                    acc.at[pl.ds(zstart + (WPT // CHUNK) * CHUNK, zrem)])
    plsc.subcore_barrier()

    for k in range(NSUB):
        pltpu.async_copy(gbuf.at[0, pl.ds(k * SUB, SUB)],
                         acc.at[lidx.at[1, k]], sem_s, add=True)
    wait_edges(0)
    fire_gather(0, 0)

    def outer(g, _):
        for j in range(4):
            i = g * 4 + j
            b = j % 2
            wait_edges((j + 1) % 4)
            fire_edges(i + 2, (j + 2) % 4)
            for m in range(GROUPS):
                r16 = rows4[j, pl.ds(m * L, L)]
                l16 = r16 - base_half
                oob = (l16 < 0) | (l16 >= HALF)
                l16 = jnp.where(oob, DUMP, l16)
                lidx[b, m // (SUB // L), pl.ds((m % (SUB // L)) * L, L)] = l16
            wait_gather(j, b)
            def scale16(m, _):
                r0 = m * L
                v16 = vals4[j, pl.ds(r0, L)]
                for ii in range(L):
                    v = v16[ii]
                    for q in range(D // L):
                        gbuf[b, r0 + ii, pl.ds(q * L, L)] = (
                            gbuf[b, r0 + ii, pl.ds(q * L, L)] * v)
                return 0
            lax.fori_loop(0, GROUPS, scale16, 0)
            wait_scatter(1 - b)
            fire_gather((j + 1) % 4, 1 - b)
            fire_scatter(b)
        return 0

    lax.fori_loop(0, OUTER, outer, 0)

    wait_edges((N_CHUNKS + 1) % 4)
    wait_gather(N_CHUNKS % 4, N_CHUNKS % 2)
    wait_scatter((N_CHUNKS - 1) % 2)
    plsc.subcore_barrier()

    wstart = s * WPT

    @pl.when(s < NS - 1)
    def _():
        for k in range(WPT // CHUNK):
            off = wstart + k * CHUNK
            pltpu.sync_copy(acc.at[pl.ds(off, CHUNK)],
                            y_hbm.at[pl.ds(base_half + off, CHUNK)])
        toff = wstart + (WPT // CHUNK) * CHUNK
        pltpu.sync_copy(acc.at[pl.ds(toff, WPT % CHUNK)],
                        y_hbm.at[pl.ds(base_half + toff, WPT % CHUNK)])

    @pl.when(s == NS - 1)
    def _():
        for k in range(W_LAST // CHUNK):
            off = wstart + k * CHUNK
            pltpu.sync_copy(acc.at[pl.ds(off, CHUNK)],
                            y_hbm.at[pl.ds(base_half + off, CHUNK)])
        toff = wstart + (W_LAST // CHUNK) * CHUNK
        pltpu.sync_copy(acc.at[pl.ds(toff, W_LAST % CHUNK)],
                        y_hbm.at[pl.ds(base_half + toff, W_LAST % CHUNK)])


_mesh = plsc.VectorSubcoreMesh(core_axis_name="c", subcore_axis_name="s")

_hop = functools.partial(
    pl.kernel,
    mesh=_mesh,
    compiler_params=pltpu.CompilerParams(use_tc_tiling_on_sc=False),
    out_type=jax.ShapeDtypeStruct((N_NODES, D), jnp.float32),
    scratch_types=[
        pltpu.VMEM_SHARED((ACC_ROWS, D), jnp.float32),
        pltpu.VMEM((2, CHUNK, D), jnp.float32),
        pltpu.VMEM((4, CHUNK), jnp.int32),
        pltpu.VMEM((4, CHUNK), jnp.int32),
        pltpu.VMEM((4, CHUNK), jnp.float32),
        pltpu.VMEM((2, NSUB, SUB), jnp.int32),
        pltpu.SemaphoreType.DMA,
        pltpu.SemaphoreType.DMA,
        pltpu.SemaphoreType.DMA,
    ],
)(_hop_kernel)


def kernel(user_embed, item_embed, edge_rows, edge_cols, edge_vals):
    x0 = jnp.concatenate([user_embed, item_embed], axis=0)
    pad = E_ALLOC - N_EDGES
    rows_p = jnp.concatenate([edge_rows, jnp.zeros((pad,), jnp.int32)])
    cols_p = jnp.concatenate([edge_cols, jnp.zeros((pad,), jnp.int32)])
    vals_p = jnp.concatenate([edge_vals, jnp.zeros((pad,), jnp.float32)])

    x1 = _hop(x0, rows_p, cols_p, vals_p)
    x2 = _hop(x1, rows_p, cols_p, vals_p)
    x3 = _hop(x2, rows_p, cols_p, vals_p)
    embs = jnp.stack([x0, x1, x2, x3], axis=1)
    return (embs[:N_USERS], embs[N_USERS:])
